# baseline (speedup 1.0000x reference)
import functools

import jax
import jax.numpy as jnp
from jax import lax
from jax.experimental import pallas as pl
from jax.experimental.pallas import tpu as pltpu
from jax.experimental.pallas import tpu_sc as plsc

N = 10000
E = 320000
D = 128

NC = 2
NS = 16
L = 16
DH = D // NC

CH = 128
NBUF = 4
G = 32
NGROUP = 5
NCHUNK = G * NGROUP
EPT_PAD = NCHUNK * CH
E_PAD = EPT_PAD * NS

ACC_ROWS = 10240
ROWS_PER_TILE = ACC_ROWS // NS
XROWS_PER_TILE = N // NS


def _sc_aggregate(x, src_t, dst_t):
    mesh = plsc.VectorSubcoreMesh(core_axis_name="c", subcore_axis_name="s")

    @functools.partial(
        pl.kernel,
        out_type=jax.ShapeDtypeStruct((NC, ACC_ROWS, DH), jnp.float32),
        mesh=mesh,
        compiler_params=pltpu.CompilerParams(use_tc_tiling_on_sc=False),
        scratch_types=[
            [pltpu.VMEM((G, CH), jnp.int32)] * 2,
            [pltpu.VMEM((G, CH), jnp.int32)] * 2,
            [pltpu.VMEM((CH, DH), jnp.float32)] * NBUF,
            pltpu.VMEM_SHARED((N, DH), jnp.float32),
            pltpu.VMEM_SHARED((ACC_ROWS, DH), jnp.float32),
            [pltpu.SemaphoreType.DMA] * NBUF,
            [pltpu.SemaphoreType.DMA] * NBUF,
            [pltpu.SemaphoreType.DMA] * 2,
        ],
    )
    def agg(x_hbm, src_hbm, dst_hbm, out_hbm, idx_sb, idx_db, bufs, xs, acc,
            gsem, ssem, isem):
        c = lax.axis_index("c")
        s = lax.axis_index("s")

        stage = pltpu.async_copy(
            x_hbm.at[pl.ds(s * XROWS_PER_TILE, XROWS_PER_TILE), pl.ds(c * DH, DH)],
            xs.at[pl.ds(s * XROWS_PER_TILE, XROWS_PER_TILE)],
            isem[0],
        )

        zero16 = jnp.zeros((L,), jnp.float32)

        def zrow(i, carry):
            for j in range(DH // L):
                bufs[0][i, pl.ds(j * L, L)] = zero16
            return carry

        lax.fori_loop(0, CH, zrow, 0, unroll=False)

        @pl.when(s < NS - 1)
        def _init_full():
            pltpu.sync_copy(
                x_hbm.at[pl.ds(s * ROWS_PER_TILE, ROWS_PER_TILE), pl.ds(c * DH, DH)],
                acc.at[pl.ds(s * ROWS_PER_TILE, ROWS_PER_TILE)],
            )

        @pl.when(s == NS - 1)
        def _init_last():
            pltpu.sync_copy(
                x_hbm.at[pl.ds((NS - 1) * ROWS_PER_TILE, N - (NS - 1) * ROWS_PER_TILE),
                         pl.ds(c * DH, DH)],
                acc.at[pl.ds((NS - 1) * ROWS_PER_TILE, N - (NS - 1) * ROWS_PER_TILE)],
            )
            pltpu.sync_copy(bufs[0], acc.at[pl.ds(N, CH)])
            pltpu.sync_copy(
                bufs[0].at[pl.ds(0, ACC_ROWS - N - CH)],
                acc.at[pl.ds(N + CH, ACC_ROWS - N - CH)],
            )

        stage.wait()
        plsc.subcore_barrier()

        GA = 2

        def fire_idx(g, p):
            pltpu.async_copy(src_hbm.at[s].at[pl.ds(g * G, G)], idx_sb[p], isem[p])
            pltpu.async_copy(dst_hbm.at[s].at[pl.ds(g * G, G)], idx_db[p], isem[p])

        def wait_idx(g, p):
            pltpu.make_async_copy(src_hbm.at[s].at[pl.ds(g * G, G)], idx_sb[p], isem[p]).wait()
            pltpu.make_async_copy(dst_hbm.at[s].at[pl.ds(g * G, G)], idx_db[p], isem[p]).wait()

        fire_idx(0, 0)
        for g in range(NGROUP):
            p = g % 2
            idx_s = idx_sb[p]
            idx_d = idx_db[p]
            wait_idx(g, p)
            if g + 1 < NGROUP:
                fire_idx(g + 1, (g + 1) % 2)

            def wait_gather(j, b):
                pltpu.make_async_copy(xs.at[idx_s.at[j]], bufs[b], gsem[b]).wait()

            def fire_scatter(j, b):
                pltpu.async_copy(bufs[b], acc.at[idx_d.at[j]], ssem[b], add=True)

            def wait_scatter(j, b):
                pltpu.make_async_copy(bufs[b], acc.at[idx_d.at[j]], ssem[b]).wait()

            def fire_gather(j, b):
                pltpu.async_copy(xs.at[idx_s.at[j]], bufs[b], gsem[b])

            for b in range(GA):
                fire_gather(b, b)

            for b in range(NBUF):
                wait_gather(b, b)
                fire_scatter(b, b)
                if b >= GA:
                    wait_scatter(b - GA, (b + GA) % NBUF)
                fire_gather(b + GA, (b + GA) % NBUF)

            def step(i, carry2):
                j0 = NBUF + i * NBUF
                for b in range(NBUF):
                    j = j0 + b
                    wait_gather(j, b)
                    fire_scatter(j, b)
                    wait_scatter(j - GA, (b + GA) % NBUF)
                    fire_gather(j + GA, (b + GA) % NBUF)
                return carry2

            lax.fori_loop(0, (G - 2 * NBUF) // NBUF, step, 0, unroll=False)

            for b in range(NBUF):
                j = G - NBUF + b
                wait_gather(j, b)
                fire_scatter(j, b)
                wait_scatter(j - GA, (b + GA) % NBUF)
                if j + GA < G:
                    fire_gather(j + GA, (b + GA) % NBUF)
            for b in range(NBUF - GA, NBUF):
                wait_scatter(G - NBUF + b, b)

        plsc.subcore_barrier()

        pltpu.sync_copy(
            acc.at[pl.ds(s * ROWS_PER_TILE, ROWS_PER_TILE)],
            out_hbm.at[c].at[pl.ds(s * ROWS_PER_TILE, ROWS_PER_TILE)],
        )

    return agg(x, src_t, dst_t)


def _tc_finish(partials, W):
    BR = 1000

    def body(p_ref, w_ref, o_ref):
        sm = jnp.concatenate([p_ref[0], p_ref[1]], axis=1)
        o_ref[...] = lax.dot_general(
            sm, w_ref[...], (((1,), (1,)), ((), ())),
            preferred_element_type=jnp.float32,
        )

    return pl.pallas_call(
        body,
        grid=(N // BR,),
        in_specs=[
            pl.BlockSpec((NC, BR, DH), lambda i: (0, i, 0)),
            pl.BlockSpec((D, D), lambda i: (0, 0)),
        ],
        out_specs=pl.BlockSpec((BR, D), lambda i: (i, 0)),
        out_shape=jax.ShapeDtypeStruct((N, D), jnp.float32),
    )(partials, W)


def kernel(x, edge_index, W):
    src = edge_index[0]
    dst = edge_index[1]
    pad = E_PAD - E
    src_p = jnp.concatenate([src, jnp.zeros((pad,), jnp.int32)])
    dst_p = jnp.concatenate([dst, jnp.full((pad,), N, jnp.int32)])
    src_t = src_p.reshape(NS, NCHUNK, CH)
    dst_t = dst_p.reshape(NS, NCHUNK, CH)

    partials = _sc_aggregate(x, src_t, dst_t)
    return _tc_finish(partials, W)


# speedup vs baseline: 1.0335x; 1.0335x over previous
import functools

import jax
import jax.numpy as jnp
from jax import lax
from jax.experimental import pallas as pl
from jax.experimental.pallas import tpu as pltpu
from jax.experimental.pallas import tpu_sc as plsc

N = 10000
E = 320000
D = 128

NC = 2
NS = 16
L = 16
DH = D // NC

CH = 128
NBUF = 4
G = 32
NGROUP = 5
NCHUNK = G * NGROUP
EPT_PAD = NCHUNK * CH
E_PAD = EPT_PAD * NS

ACC_ROWS = 10240
ROWS_PER_TILE = ACC_ROWS // NS
XROWS_PER_TILE = N // NS


def _sc_aggregate(x, src_t, dst_t):
    mesh = plsc.VectorSubcoreMesh(core_axis_name="c", subcore_axis_name="s")

    @functools.partial(
        pl.kernel,
        out_type=jax.ShapeDtypeStruct((NC, ACC_ROWS, DH), jnp.float32),
        mesh=mesh,
        compiler_params=pltpu.CompilerParams(use_tc_tiling_on_sc=False),
        scratch_types=[
            [pltpu.VMEM((G, CH), jnp.int32)] * 2,
            [pltpu.VMEM((G, CH), jnp.int32)] * 2,
            [pltpu.VMEM((CH, DH), jnp.float32)] * NBUF,
            pltpu.VMEM_SHARED((N, DH), jnp.float32),
            pltpu.VMEM_SHARED((ACC_ROWS, DH), jnp.float32),
            [pltpu.SemaphoreType.DMA] * NBUF,
            [pltpu.SemaphoreType.DMA] * NBUF,
            [pltpu.SemaphoreType.DMA] * 2,
        ],
    )
    def agg(x_hbm, src_hbm, dst_hbm, out_hbm, idx_sb, idx_db, bufs, xs, acc,
            gsem, ssem, isem):
        c = lax.axis_index("c")
        s = lax.axis_index("s")

        stage = pltpu.async_copy(
            x_hbm.at[pl.ds(s * XROWS_PER_TILE, XROWS_PER_TILE), pl.ds(c * DH, DH)],
            xs.at[pl.ds(s * XROWS_PER_TILE, XROWS_PER_TILE)],
            isem[0],
        )

        zero16 = jnp.zeros((L,), jnp.float32)

        def zrow(i, carry):
            for j in range(DH // L):
                bufs[0][i, pl.ds(j * L, L)] = zero16
            return carry

        lax.fori_loop(0, CH, zrow, 0, unroll=False)

        def zacc(k, carry):
            pltpu.sync_copy(bufs[0], acc.at[pl.ds(s * ROWS_PER_TILE + k * CH, CH)])
            return carry

        lax.fori_loop(0, ROWS_PER_TILE // CH, zacc, 0, unroll=False)

        stage.wait()
        plsc.subcore_barrier()

        GA = 2

        def fire_idx(g, p):
            pltpu.async_copy(src_hbm.at[s].at[pl.ds(g * G, G)], idx_sb[p], isem[p])
            pltpu.async_copy(dst_hbm.at[s].at[pl.ds(g * G, G)], idx_db[p], isem[p])

        def wait_idx(g, p):
            pltpu.make_async_copy(src_hbm.at[s].at[pl.ds(g * G, G)], idx_sb[p], isem[p]).wait()
            pltpu.make_async_copy(dst_hbm.at[s].at[pl.ds(g * G, G)], idx_db[p], isem[p]).wait()

        fire_idx(0, 0)
        for g in range(NGROUP):
            p = g % 2
            idx_s = idx_sb[p]
            idx_d = idx_db[p]
            wait_idx(g, p)
            if g + 1 < NGROUP:
                fire_idx(g + 1, (g + 1) % 2)

            def wait_gather(j, b):
                pltpu.make_async_copy(xs.at[idx_s.at[j]], bufs[b], gsem[b]).wait()

            def fire_scatter(j, b):
                pltpu.async_copy(bufs[b], acc.at[idx_d.at[j]], ssem[b], add=True)

            def wait_scatter(j, b):
                pltpu.make_async_copy(bufs[b], acc.at[idx_d.at[j]], ssem[b]).wait()

            def fire_gather(j, b):
                pltpu.async_copy(xs.at[idx_s.at[j]], bufs[b], gsem[b])

            for b in range(GA):
                fire_gather(b, b)

            for b in range(NBUF):
                wait_gather(b, b)
                fire_scatter(b, b)
                if b >= GA:
                    wait_scatter(b - GA, (b + GA) % NBUF)
                fire_gather(b + GA, (b + GA) % NBUF)

            def step(i, carry2):
                j0 = NBUF + i * NBUF
                for b in range(NBUF):
                    j = j0 + b
                    wait_gather(j, b)
                    fire_scatter(j, b)
                    wait_scatter(j - GA, (b + GA) % NBUF)
                    fire_gather(j + GA, (b + GA) % NBUF)
                return carry2

            lax.fori_loop(0, (G - 2 * NBUF) // NBUF, step, 0, unroll=False)

            for b in range(NBUF):
                j = G - NBUF + b
                wait_gather(j, b)
                fire_scatter(j, b)
                wait_scatter(j - GA, (b + GA) % NBUF)
                if j + GA < G:
                    fire_gather(j + GA, (b + GA) % NBUF)
            for b in range(NBUF - GA, NBUF):
                wait_scatter(G - NBUF + b, b)

        plsc.subcore_barrier()

        pltpu.sync_copy(
            acc.at[pl.ds(s * ROWS_PER_TILE, ROWS_PER_TILE)],
            out_hbm.at[c].at[pl.ds(s * ROWS_PER_TILE, ROWS_PER_TILE)],
        )

    return agg(x, src_t, dst_t)


def _tc_finish(partials, x, W):
    BR = 2000

    def body(p_ref, x_ref, w_ref, o_ref):
        sm = jnp.concatenate([p_ref[0], p_ref[1]], axis=1) + x_ref[...]
        o_ref[...] = lax.dot_general(
            sm, w_ref[...], (((1,), (1,)), ((), ())),
            preferred_element_type=jnp.float32,
        )

    return pl.pallas_call(
        body,
        grid=(N // BR,),
        in_specs=[
            pl.BlockSpec((NC, BR, DH), lambda i: (0, i, 0)),
            pl.BlockSpec((BR, D), lambda i: (i, 0)),
            pl.BlockSpec((D, D), lambda i: (0, 0)),
        ],
        out_specs=pl.BlockSpec((BR, D), lambda i: (i, 0)),
        out_shape=jax.ShapeDtypeStruct((N, D), jnp.float32),
    )(partials, x, W)


def kernel(x, edge_index, W):
    src = edge_index[0]
    dst = edge_index[1]
    pad = E_PAD - E
    src_p = jnp.concatenate([src, jnp.zeros((pad,), jnp.int32)])
    dst_p = jnp.concatenate([dst, jnp.full((pad,), N, jnp.int32)])
    src_t = src_p.reshape(NS, NCHUNK, CH)
    dst_t = dst_p.reshape(NS, NCHUNK, CH)

    partials = _sc_aggregate(x, src_t, dst_t)
    return _tc_finish(partials, x, W)
